# Initial kernel scaffold; baseline (speedup 1.0000x reference)
import functools

import jax
import jax.numpy as jnp
import numpy as np
from jax import lax
from jax.experimental import pallas as pl
from jax.experimental.pallas import tpu as pltpu
from jax.experimental.pallas import tpu_sc as plsc

_N = 10000
_E = 320000
_H = 4
_HID = 128

_CH = 80
_NTILES = 32
_EPT = _E // _NTILES
_NCH = _EPT // _CH
_ZROWS = 1000

_f32 = jnp.float32
_i32 = jnp.int32


def _mesh():
    return plsc.VectorSubcoreMesh(core_axis_name="c", subcore_axis_name="s")


def _lane_patterns():
    l = lax.iota(_i32, 16)
    return l // 8, l % 8, l % 4


def _pass_a_body(src_hbm, dst_hbm, aa_hbm, mv_hbm, z8_hbm,
                 ealpha_out, dpart_out,
                 denom_sh, srcv, dstv, gsrc, gdst, e8, mvv, sem1, sem2):
    c = lax.axis_index("c")
    s = lax.axis_index("s")
    wid = c * 16 + s

    @pl.when(s < _N // _ZROWS)
    def _():
        pltpu.sync_copy(z8_hbm.at[pl.ds(s * _ZROWS, _ZROWS)],
                        denom_sh.at[pl.ds(s * _ZROWS, _ZROWS)])

    pltpu.sync_copy(mv_hbm, mvv)
    plsc.subcore_barrier()

    rowo, col8, col4 = _lane_patterns()
    base = wid * _EPT
    mv = mvv[...]

    def chunk(i, carry):
        eb = base + i * _CH
        pltpu.sync_copy(src_hbm.at[pl.ds(eb, _CH)], srcv)
        pltpu.sync_copy(dst_hbm.at[pl.ds(eb, _CH)], dstv)
        cp1 = pltpu.async_copy(aa_hbm.at[srcv], gsrc, sem1)
        cp2 = pltpu.async_copy(aa_hbm.at[dstv], gdst, sem2)
        cp1.wait()
        cp2.wait()
        for m in range(_CH // 2):
            row = 2 * m + rowo
            a_s = plsc.load_gather(gsrc, [row, col4])
            a_d = plsc.load_gather(gdst, [row, col4 + 4])
            a = a_s + a_d
            a = jnp.maximum(a, a * 0.2)
            e = jnp.exp(a - mv)
            plsc.store_scatter(e8, [row, col8], e)
        pltpu.sync_copy(e8, ealpha_out.at[pl.ds(eb, _CH)])
        pltpu.sync_copy(e8, denom_sh.at[dstv], add=True)
        return carry

    lax.fori_loop(0, _NCH, chunk, 0)
    plsc.subcore_barrier()

    @pl.when(s < _N // _ZROWS)
    def _():
        pltpu.sync_copy(denom_sh.at[pl.ds(s * _ZROWS, _ZROWS)],
                        dpart_out.at[c, pl.ds(s * _ZROWS, _ZROWS)])


def _make_pass_a():
    return functools.partial(
        pl.kernel,
        out_type=(jax.ShapeDtypeStruct((_E, 8), _f32),
                  jax.ShapeDtypeStruct((2, _N, 8), _f32)),
        mesh=_mesh(),
        scratch_types=(
            pltpu.VMEM_SHARED((_N, 8), _f32),
            pltpu.VMEM((_CH,), _i32),
            pltpu.VMEM((_CH,), _i32),
            pltpu.VMEM((_CH, 8), _f32),
            pltpu.VMEM((_CH, 8), _f32),
            pltpu.VMEM((_CH, 8), _f32),
            pltpu.VMEM((16,), _f32),
            pltpu.SemaphoreType.DMA,
            pltpu.SemaphoreType.DMA,
        ),
    )(_pass_a_body)


def _pass_b_common(src_hbm, dst_hbm, e_hbm, r_hbm,
                   acc_sh, z128_hbm, acc_out, srcv, dstv, e8, r8, w8,
                   sem2, start_gather, compute):
    c = lax.axis_index("c")
    s = lax.axis_index("s")
    wid = c * 16 + s

    @pl.when(s < _N // _ZROWS)
    def _():
        pltpu.sync_copy(z128_hbm.at[pl.ds(s * _ZROWS, _ZROWS)],
                        acc_sh.at[pl.ds(s * _ZROWS, _ZROWS)])

    plsc.subcore_barrier()

    rowo, col8, _ = _lane_patterns()
    base = wid * _EPT

    def chunk(i, carry):
        eb = base + i * _CH
        pltpu.sync_copy(src_hbm.at[pl.ds(eb, _CH)], srcv)
        pltpu.sync_copy(dst_hbm.at[pl.ds(eb, _CH)], dstv)
        gather = start_gather(srcv)
        cp2 = pltpu.async_copy(r_hbm.at[dstv], r8, sem2)
        pltpu.sync_copy(e_hbm.at[pl.ds(eb, _CH)], e8)
        cp2.wait()
        for m in range(_CH // 2):
            row = 2 * m + rowo
            ee = plsc.load_gather(e8, [row, col8])
            rr = plsc.load_gather(r8, [row, col8])
            plsc.store_scatter(w8, [row, col8], ee * rr)
        gather.wait()
        msg = compute()
        pltpu.sync_copy(msg, acc_sh.at[dstv], add=True)
        return carry

    lax.fori_loop(0, _NCH, chunk, 0)
    plsc.subcore_barrier()

    @pl.when(s < _N // _ZROWS)
    def _():
        pltpu.sync_copy(acc_sh.at[pl.ds(s * _ZROWS, _ZROWS)],
                        acc_out.at[c, pl.ds(s * _ZROWS, _ZROWS)])


def _pass_b12_body(src_hbm, dst_hbm, e_hbm, r_hbm, h_hbm, z128_hbm,
                   acc_out,
                   acc_sh, srcv, dstv, e8, r8, w8, rows, sem1, sem2):
    def start_gather(srcv_):
        return pltpu.async_copy(h_hbm.at[srcv_], rows, sem1)

    def compute():
        def rowfn(j, carry):
            for k in range(8):
                wsc = w8[j, k // 2]
                rows[j, pl.ds(16 * k, 16)] = rows[j, pl.ds(16 * k, 16)] * wsc
            return carry
        lax.fori_loop(0, _CH, rowfn, 0)
        return rows

    _pass_b_common(src_hbm, dst_hbm, e_hbm, r_hbm, acc_sh, z128_hbm, acc_out,
                   srcv, dstv, e8, r8, w8, sem2, start_gather, compute)


def _pass_b3_body(src_hbm, dst_hbm, e_hbm, r_hbm, h_hbm, z128_hbm,
                  acc_out,
                  acc_sh, srcv, dstv, e8, r8, w8, rows, msg, sem1, sem2):
    def start_gather(srcv_):
        return pltpu.async_copy(h_hbm.at[srcv_], rows, sem1)

    def compute():
        def rowfn(j, carry):
            for k in range(8):
                acc = rows[j, pl.ds(16 * k, 16)] * w8[j, 0]
                for hh in range(1, _H):
                    acc = acc + rows[j, pl.ds(128 * hh + 16 * k, 16)] * w8[j, hh]
                msg[j, pl.ds(16 * k, 16)] = acc
            return carry
        lax.fori_loop(0, _CH, rowfn, 0)
        return msg

    _pass_b_common(src_hbm, dst_hbm, e_hbm, r_hbm, acc_sh, z128_hbm, acc_out,
                   srcv, dstv, e8, r8, w8, sem2, start_gather, compute)


def _make_pass_b(hdim):
    body = _pass_b12_body if hdim == _HID else _pass_b3_body
    scratch = [
        pltpu.VMEM_SHARED((_N, _HID), _f32),
        pltpu.VMEM((_CH,), _i32),
        pltpu.VMEM((_CH,), _i32),
        pltpu.VMEM((_CH, 8), _f32),
        pltpu.VMEM((_CH, 8), _f32),
        pltpu.VMEM((_CH, 8), _f32),
        pltpu.VMEM((_CH, hdim), _f32),
    ]
    if hdim != _HID:
        scratch.append(pltpu.VMEM((_CH, _HID), _f32))
    scratch += [pltpu.SemaphoreType.DMA, pltpu.SemaphoreType.DMA]
    return functools.partial(
        pl.kernel,
        out_type=jax.ShapeDtypeStruct((2, _N, _HID), _f32),
        mesh=_mesh(),
        scratch_types=tuple(scratch),
    )(body)


def _tc_pre1_body(x_ref, w_ref, o_ref, av_ref, h_out, aa_out, m_out):
    hh = jnp.dot(x_ref[...], w_ref[...], preferred_element_type=_f32)
    h_out[...] = hh
    aa = jnp.dot(hh, o_ref[...] * av_ref[...], preferred_element_type=_f32)
    aa_out[...] = aa
    m8 = jnp.max(aa, axis=0, keepdims=True)
    mh = m8[:, 0:4] + m8[:, 4:8]
    mh = jnp.maximum(mh, mh * 0.2)
    m_out[...] = jnp.concatenate([mh, mh, mh, mh], axis=1)


def _tc_pre23_body(ap_ref, b_ref, g_ref, be_ref, w_ref, o_ref, av_ref,
                   h_out, aa_out, m_out):
    y = ap_ref[0] + ap_ref[1] + b_ref[...]
    mean = jnp.mean(y, axis=0, keepdims=True)
    var = jnp.mean(y * y, axis=0, keepdims=True) - mean * mean
    xn = (y - mean) * lax.rsqrt(var + 1e-5) * g_ref[...] + be_ref[...]
    x2 = jnp.where(xn > 0, xn, jnp.exp(jnp.minimum(xn, 0.0)) - 1.0)
    hh = jnp.dot(x2, w_ref[...], preferred_element_type=_f32)
    h_out[...] = hh
    aa = jnp.dot(hh, o_ref[...] * av_ref[...], preferred_element_type=_f32)
    aa_out[...] = aa
    m8 = jnp.max(aa, axis=0, keepdims=True)
    mh = m8[:, 0:4] + m8[:, 4:8]
    mh = jnp.maximum(mh, mh * 0.2)
    m_out[...] = jnp.concatenate([mh, mh, mh, mh], axis=1)


def _mk_mid(scale):
    def body(dp_ref, r_out):
        d = dp_ref[0] + dp_ref[1]
        r_out[...] = scale / (d + 1e-16)
    return pl.pallas_call(
        body, out_shape=jax.ShapeDtypeStruct((_N, 8), _f32))


def _tc_final_body(ap_ref, b_ref, g_ref, be_ref, w1_ref, b1_ref, w2_ref,
                   b2_ref, out_ref):
    y = ap_ref[0] + ap_ref[1] + b_ref[...]
    mean = jnp.mean(y, axis=0, keepdims=True)
    var = jnp.mean(y * y, axis=0, keepdims=True) - mean * mean
    xn = (y - mean) * lax.rsqrt(var + 1e-5) * g_ref[...] + be_ref[...]
    t = jnp.dot(xn, w1_ref[...], preferred_element_type=_f32) + b1_ref[...]
    t = jnp.where(t > 0, t, jnp.exp(jnp.minimum(t, 0.0)) - 1.0)
    out_ref[...] = jnp.sum(t * w2_ref[...], axis=1, keepdims=True) + b2_ref[...]


def _tc_pre1(hdim):
    return pl.pallas_call(
        _tc_pre1_body,
        out_shape=(jax.ShapeDtypeStruct((_N, hdim), _f32),
                   jax.ShapeDtypeStruct((_N, 8), _f32),
                   jax.ShapeDtypeStruct((1, 16), _f32)))


def _tc_pre23(hdim):
    return pl.pallas_call(
        _tc_pre23_body,
        out_shape=(jax.ShapeDtypeStruct((_N, hdim), _f32),
                   jax.ShapeDtypeStruct((_N, 8), _f32),
                   jax.ShapeDtypeStruct((1, 16), _f32)))


_tc_final = pl.pallas_call(
    _tc_final_body, out_shape=jax.ShapeDtypeStruct((_N, 1), _f32))


def _onehot8(hc, ch):
    i = np.arange(hc)[:, None]
    j = np.arange(8)[None, :]
    return jnp.asarray(((i // ch) == (j % 4)).astype(np.float32))


_O128 = _onehot8(128, 32)
_O512 = _onehot8(512, 128)


def _av8(att_s, att_d):
    asf = att_s.reshape(-1, 1)
    adf = att_d.reshape(-1, 1)
    return jnp.concatenate(
        [jnp.broadcast_to(asf, (asf.shape[0], 4)),
         jnp.broadcast_to(adf, (adf.shape[0], 4))], axis=1)


def kernel(x, edge_index, W1, as1, ad1, b1, g1, be1, W2, as2, ad2, b2, g2,
           be2, W3, as3, ad3, b3, g3, be3, Wc1, bc1, Wc2, bc2):
    src = edge_index[0].astype(_i32)
    dst = edge_index[1].astype(_i32)
    z8 = jnp.zeros((_N, 8), _f32)
    z128 = jnp.zeros((_N, _HID), _f32)

    pass_a = _make_pass_a()
    pass_b12 = _make_pass_b(_HID)
    pass_b3 = _make_pass_b(_H * _HID)
    mid1 = _mk_mid(1.0)
    mid3 = _mk_mid(1.0 / _H)

    h1, aa1, m1 = _tc_pre1(_HID)(x, W1, _O128, _av8(as1, ad1))
    e1, dp1 = pass_a(src, dst, aa1, m1.reshape(16), z8)
    r1 = mid1(dp1)
    ap1 = pass_b12(src, dst, e1, r1, h1, z128)

    h2, aa2, m2 = _tc_pre23(_HID)(
        ap1, b1.reshape(1, -1), g1.reshape(1, -1), be1.reshape(1, -1),
        W2, _O128, _av8(as2, ad2))
    e2, dp2 = pass_a(src, dst, aa2, m2.reshape(16), z8)
    r2 = mid1(dp2)
    ap2 = pass_b12(src, dst, e2, r2, h2, z128)

    h3, aa3, m3 = _tc_pre23(_H * _HID)(
        ap2, b2.reshape(1, -1), g2.reshape(1, -1), be2.reshape(1, -1),
        W3, _O512, _av8(as3, ad3))
    e3, dp3 = pass_a(src, dst, aa3, m3.reshape(16), z8)
    r3 = mid3(dp3)
    ap3 = pass_b3(src, dst, e3, r3, h3, z128)

    out = _tc_final(
        ap3, b3.reshape(1, -1), g3.reshape(1, -1), be3.reshape(1, -1),
        Wc1, bc1.reshape(1, -1), Wc2.reshape(1, -1), bc2.reshape(1, 1))
    return out.reshape(_N)


# speedup vs baseline: 25.1835x; 25.1835x over previous
import functools

import jax
import jax.numpy as jnp
import numpy as np
from jax import lax
from jax.experimental import pallas as pl
from jax.experimental.pallas import tpu as pltpu
from jax.experimental.pallas import tpu_sc as plsc

_N = 10000
_E = 320000
_H = 4
_HID = 128

_CH = 80
_CH3 = 40
_NTILES = 32
_EPT = _E // _NTILES
_ZROWS = 1000

_f32 = jnp.float32
_i32 = jnp.int32


def _mesh():
    return plsc.VectorSubcoreMesh(core_axis_name="c", subcore_axis_name="s",
                                  num_cores=2, num_subcores=16)


_SC_PARAMS = pltpu.CompilerParams(use_tc_tiling_on_sc=False,
                                  needs_layout_passes=False,
                                  has_side_effects=True)


def _lane_patterns():
    l = lax.iota(_i32, 16)
    return l // 8, l % 8, l % 4


def _pass_a_body(src_hbm, dst_hbm, aa_hbm, mv_hbm, z8_hbm,
                 ealpha_out, dpart_out,
                 denom_sh, srcv, dstv, gsrc, gdst, e8f, e82, mvv, sem1, sem2):
    c = lax.axis_index("c")
    s = lax.axis_index("s")
    wid = c * 16 + s

    @pl.when(s < _N // _ZROWS)
    def _():
        pltpu.sync_copy(z8_hbm.at[pl.ds(s * _ZROWS, _ZROWS)],
                        denom_sh.at[pl.ds(s * _ZROWS, _ZROWS)])

    pltpu.sync_copy(mv_hbm, mvv)
    plsc.subcore_barrier()

    base = wid * _EPT

    def chunk(i, carry):
        rowo, col8, col4 = _lane_patterns()
        mv = mvv[...]
        eb = base + i * _CH
        pltpu.sync_copy(src_hbm.at[pl.ds(eb, _CH)], srcv)
        pltpu.sync_copy(dst_hbm.at[pl.ds(eb, _CH)], dstv)
        cp1 = pltpu.async_copy(aa_hbm.at[srcv], gsrc, sem1)
        cp2 = pltpu.async_copy(aa_hbm.at[dstv], gdst, sem2)
        cp1.wait()
        cp2.wait()
        for m in range(_CH // 2):
            row = 2 * m + rowo
            a_s = plsc.load_gather(gsrc, [row, col4])
            a_d = plsc.load_gather(gdst, [row, col4 + 4])
            a = a_s + a_d
            a = jnp.maximum(a, a * 0.2)
            e = jnp.exp(a - mv)
            e8f[pl.ds(16 * m, 16)] = e
            plsc.store_scatter(e82, [row, col8], e)
        pltpu.sync_copy(e8f.at[pl.ds(0, _CH * 8)],
                        ealpha_out.at[pl.ds(eb * 8, _CH * 8)])
        pltpu.sync_copy(e82, denom_sh.at[dstv], add=True)
        return carry

    lax.fori_loop(0, _EPT // _CH, chunk, 0)
    plsc.subcore_barrier()

    @pl.when(s < _N // _ZROWS)
    def _():
        pltpu.sync_copy(denom_sh.at[pl.ds(s * _ZROWS, _ZROWS)],
                        dpart_out.at[c, pl.ds(s * _ZROWS, _ZROWS)])


def _make_pass_a():
    return functools.partial(
        pl.kernel,
        out_type=(jax.ShapeDtypeStruct((_E * 8,), _f32),
                  jax.ShapeDtypeStruct((2, _N, 8), _f32)),
        mesh=_mesh(),
        compiler_params=_SC_PARAMS,
        scratch_types=(
            pltpu.VMEM_SHARED((_N, 8), _f32),
            pltpu.VMEM((_CH,), _i32),
            pltpu.VMEM((_CH,), _i32),
            pltpu.VMEM((_CH, 8), _f32),
            pltpu.VMEM((_CH, 8), _f32),
            pltpu.VMEM((_CH * 8,), _f32),
            pltpu.VMEM((_CH, 8), _f32),
            pltpu.VMEM((16,), _f32),
            pltpu.SemaphoreType.DMA,
            pltpu.SemaphoreType.DMA,
        ),
    )(_pass_a_body)


def _pass_b12_body(src_hbm, dst_hbm, e_hbm, h_hbm, z128_hbm,
                   acc_out,
                   acc_sh, srcv, dstv, wbuf, rows, sem1):
    c = lax.axis_index("c")
    s = lax.axis_index("s")
    wid = c * 16 + s

    @pl.when(s < _N // _ZROWS)
    def _():
        pltpu.sync_copy(z128_hbm.at[pl.ds(s * _ZROWS, _ZROWS)],
                        acc_sh.at[pl.ds(s * _ZROWS, _ZROWS)])

    plsc.subcore_barrier()
    base = wid * _EPT

    def chunk(i, carry):
        eb = base + i * _CH
        pltpu.sync_copy(src_hbm.at[pl.ds(eb, _CH)], srcv)
        pltpu.sync_copy(dst_hbm.at[pl.ds(eb, _CH)], dstv)
        gather = pltpu.async_copy(h_hbm.at[srcv], rows, sem1)
        pltpu.sync_copy(e_hbm.at[pl.ds(eb * 8, _CH * 8)],
                        wbuf.at[pl.ds(0, _CH * 8)])
        gather.wait()

        def rowfn(j, cc):
            wv = wbuf[pl.ds(8 * j, 16)]
            for k in range(8):
                rows[j, pl.ds(16 * k, 16)] = rows[j, pl.ds(16 * k, 16)] * wv[k // 2]
            return cc

        lax.fori_loop(0, _CH, rowfn, 0)
        pltpu.sync_copy(rows, acc_sh.at[dstv], add=True)
        return carry

    lax.fori_loop(0, _EPT // _CH, chunk, 0)
    plsc.subcore_barrier()

    @pl.when(s < _N // _ZROWS)
    def _():
        pltpu.sync_copy(acc_sh.at[pl.ds(s * _ZROWS, _ZROWS)],
                        acc_out.at[c, pl.ds(s * _ZROWS, _ZROWS)])


def _pass_b3_body(src_hbm, dst_hbm, e_hbm, r_hbm, h_hbm, z128_hbm,
                  acc_out,
                  acc_sh, srcv, dstv, ebuf, r8, wbuf, rows, msg, sem1, sem2):
    c = lax.axis_index("c")
    s = lax.axis_index("s")
    wid = c * 16 + s

    @pl.when(s < _N // _ZROWS)
    def _():
        pltpu.sync_copy(z128_hbm.at[pl.ds(s * _ZROWS, _ZROWS)],
                        acc_sh.at[pl.ds(s * _ZROWS, _ZROWS)])

    plsc.subcore_barrier()

    rowo, col8, _ = _lane_patterns()
    base = wid * _EPT

    def chunk(i, carry):
        eb = base + i * _CH3
        pltpu.sync_copy(src_hbm.at[pl.ds(eb, _CH3)], srcv)
        pltpu.sync_copy(dst_hbm.at[pl.ds(eb, _CH3)], dstv)
        gather = pltpu.async_copy(h_hbm.at[srcv], rows, sem1)
        cp2 = pltpu.async_copy(r_hbm.at[dstv], r8, sem2)
        pltpu.sync_copy(e_hbm.at[pl.ds(eb * 8, _CH3 * 8)],
                        ebuf.at[pl.ds(0, _CH3 * 8)])
        cp2.wait()
        for m in range(_CH3 // 2):
            row = 2 * m + rowo
            ee = ebuf[pl.ds(16 * m, 16)]
            rr = plsc.load_gather(r8, [row, col8])
            wbuf[pl.ds(16 * m, 16)] = ee * rr
        gather.wait()

        def rowfn(j, cc):
            wv = wbuf[pl.ds(8 * j, 16)]
            for k in range(8):
                acc = rows[j, pl.ds(16 * k, 16)] * wv[0]
                for hh in range(1, _H):
                    acc = acc + rows[j, pl.ds(128 * hh + 16 * k, 16)] * wv[hh]
                msg[j, pl.ds(16 * k, 16)] = acc
            return cc

        lax.fori_loop(0, _CH3, rowfn, 0)
        pltpu.sync_copy(msg, acc_sh.at[dstv], add=True)
        return carry

    lax.fori_loop(0, _EPT // _CH3, chunk, 0)
    plsc.subcore_barrier()

    @pl.when(s < _N // _ZROWS)
    def _():
        pltpu.sync_copy(acc_sh.at[pl.ds(s * _ZROWS, _ZROWS)],
                        acc_out.at[c, pl.ds(s * _ZROWS, _ZROWS)])


def _make_pass_b12():
    return functools.partial(
        pl.kernel,
        out_type=jax.ShapeDtypeStruct((2, _N, _HID), _f32),
        mesh=_mesh(),
        compiler_params=_SC_PARAMS,
        scratch_types=(
            pltpu.VMEM_SHARED((_N, _HID), _f32),
            pltpu.VMEM((_CH,), _i32),
            pltpu.VMEM((_CH,), _i32),
            pltpu.VMEM(((_CH + 2) * 8,), _f32),
            pltpu.VMEM((_CH, _HID), _f32),
            pltpu.SemaphoreType.DMA,
        ),
    )(_pass_b12_body)


def _make_pass_b3():
    return functools.partial(
        pl.kernel,
        out_type=jax.ShapeDtypeStruct((2, _N, _HID), _f32),
        mesh=_mesh(),
        compiler_params=_SC_PARAMS,
        scratch_types=(
            pltpu.VMEM_SHARED((_N, _HID), _f32),
            pltpu.VMEM((_CH3,), _i32),
            pltpu.VMEM((_CH3,), _i32),
            pltpu.VMEM((_CH3 * 8,), _f32),
            pltpu.VMEM((_CH3, 8), _f32),
            pltpu.VMEM(((_CH3 + 2) * 8,), _f32),
            pltpu.VMEM((_CH3, _H * _HID), _f32),
            pltpu.VMEM((_CH3, _HID), _f32),
            pltpu.SemaphoreType.DMA,
            pltpu.SemaphoreType.DMA,
        ),
    )(_pass_b3_body)


def _attention_tail(hh, o_ref, av_ref, aa_out, m_out):
    aa = jnp.dot(hh, o_ref[...] * av_ref[...], preferred_element_type=_f32)
    aa_out[...] = aa
    m8 = jnp.max(aa, axis=0, keepdims=True)
    mh = m8[:, 0:4] + m8[:, 4:8]
    mh = jnp.maximum(mh, mh * 0.2)
    m_out[...] = jnp.concatenate([mh, mh, mh, mh], axis=1)


def _tc_pre1_body(x_ref, w_ref, o_ref, av_ref, h_out, aa_out, m_out):
    hh = jnp.dot(x_ref[...], w_ref[...], preferred_element_type=_f32)
    h_out[...] = hh
    _attention_tail(hh, o_ref, av_ref, aa_out, m_out)


def _tc_pre23_body(ap_ref, dp_ref, r_ref, b_ref, g_ref, be_ref, w_ref,
                   o_ref, av_ref, h_out, aa_out, m_out):
    den = dp_ref[0] + dp_ref[1]
    rexp = jnp.dot(1.0 / (den + 1e-16), r_ref[...],
                   preferred_element_type=_f32)
    y = (ap_ref[0] + ap_ref[1]) * rexp + b_ref[...]
    mean = jnp.mean(y, axis=0, keepdims=True)
    yc = y - mean
    var = jnp.mean(yc * yc, axis=0, keepdims=True)
    xn = yc * lax.rsqrt(var + 1e-5) * g_ref[...] + be_ref[...]
    x2 = jnp.where(xn > 0, xn, jnp.exp(jnp.minimum(xn, 0.0)) - 1.0)
    hh = jnp.dot(x2, w_ref[...], preferred_element_type=_f32)
    h_out[...] = hh
    _attention_tail(hh, o_ref, av_ref, aa_out, m_out)


def _mid3_body(dp_ref, r_out):
    d = dp_ref[0] + dp_ref[1]
    r_out[...] = 0.25 / (d + 1e-16)


_mid3 = pl.pallas_call(
    _mid3_body, out_shape=jax.ShapeDtypeStruct((_N, 8), _f32))


def _tc_final_body(ap_ref, b_ref, g_ref, be_ref, w1_ref, b1_ref, w2_ref,
                   b2_ref, out_ref):
    y = ap_ref[0] + ap_ref[1] + b_ref[...]
    mean = jnp.mean(y, axis=0, keepdims=True)
    yc = y - mean
    var = jnp.mean(yc * yc, axis=0, keepdims=True)
    xn = yc * lax.rsqrt(var + 1e-5) * g_ref[...] + be_ref[...]
    t = jnp.dot(xn, w1_ref[...], preferred_element_type=_f32) + b1_ref[...]
    t = jnp.where(t > 0, t, jnp.exp(jnp.minimum(t, 0.0)) - 1.0)
    out_ref[...] = jnp.sum(t * w2_ref[...], axis=1, keepdims=True) + b2_ref[...]


def _tc_pre1(hdim):
    return pl.pallas_call(
        _tc_pre1_body,
        out_shape=(jax.ShapeDtypeStruct((_N, hdim), _f32),
                   jax.ShapeDtypeStruct((_N, 8), _f32),
                   jax.ShapeDtypeStruct((1, 16), _f32)))


def _tc_pre23(hdim):
    return pl.pallas_call(
        _tc_pre23_body,
        out_shape=(jax.ShapeDtypeStruct((_N, hdim), _f32),
                   jax.ShapeDtypeStruct((_N, 8), _f32),
                   jax.ShapeDtypeStruct((1, 16), _f32)),
        compiler_params=pltpu.CompilerParams(vmem_limit_bytes=100 * 1024 * 1024))


_tc_final = pl.pallas_call(
    _tc_final_body, out_shape=jax.ShapeDtypeStruct((_N, 1), _f32))


def _onehot8(hc, ch):
    i = np.arange(hc)[:, None]
    j = np.arange(8)[None, :]
    return ((i // ch) == (j % 4)).astype(np.float32)


_O128 = _onehot8(128, 32)
_O512 = _onehot8(512, 128)
_R8 = np.zeros((8, 128), np.float32)
for _h in range(4):
    _R8[_h, _h * 32:(_h + 1) * 32] = 1.0


def _av8(att_s, att_d):
    asf = att_s.reshape(-1, 1)
    adf = att_d.reshape(-1, 1)
    return jnp.concatenate(
        [jnp.broadcast_to(asf, (asf.shape[0], 4)),
         jnp.broadcast_to(adf, (adf.shape[0], 4))], axis=1)


def kernel(x, edge_index, W1, as1, ad1, b1, g1, be1, W2, as2, ad2, b2, g2,
           be2, W3, as3, ad3, b3, g3, be3, Wc1, bc1, Wc2, bc2):
    src = edge_index[0].astype(_i32)
    dst = edge_index[1].astype(_i32)
    z8 = jnp.zeros((_N, 8), _f32)
    z128 = jnp.zeros((_N, _HID), _f32)

    pass_a = _make_pass_a()
    pass_b12 = _make_pass_b12()
    pass_b3 = _make_pass_b3()

    h1, aa1, m1 = _tc_pre1(_HID)(x, W1, _O128, _av8(as1, ad1))
    e1, dp1 = pass_a(src, dst, aa1, m1.reshape(16), z8)
    ap1 = pass_b12(src, dst, e1, h1, z128)

    h2, aa2, m2 = _tc_pre23(_HID)(
        ap1, dp1, _R8, b1.reshape(1, -1), g1.reshape(1, -1),
        be1.reshape(1, -1), W2, _O128, _av8(as2, ad2))
    e2, dp2 = pass_a(src, dst, aa2, m2.reshape(16), z8)
    ap2 = pass_b12(src, dst, e2, h2, z128)

    h3, aa3, m3 = _tc_pre23(_H * _HID)(
        ap2, dp2, _R8, b2.reshape(1, -1), g2.reshape(1, -1),
        be2.reshape(1, -1), W3, _O512, _av8(as3, ad3))
    e3, dp3 = pass_a(src, dst, aa3, m3.reshape(16), z8)
    r3 = _mid3(dp3)
    ap3 = pass_b3(src, dst, e3, r3, h3, z128)

    out = _tc_final(
        ap3, b3.reshape(1, -1), g3.reshape(1, -1), be3.reshape(1, -1),
        Wc1, bc1.reshape(1, -1), Wc2.reshape(1, -1), bc2.reshape(1, 1))
    return out.reshape(_N)
